# baseline (speedup 1.0000x reference)
import functools

import jax
import jax.numpy as jnp
from jax import lax
from jax.experimental import pallas as pl
from jax.experimental.pallas import tpu as pltpu
from jax.experimental.pallas import tpu_sc as plsc

E = 320000
D = 128
V = 10000
TW = D // 2
NW = 32
EPW = E // NW
C = 80
NCHUNK = EPW // C
RPS = V // 16
PCH = 125
NPCH = RPS // PCH

_mesh = plsc.VectorSubcoreMesh(core_axis_name="c", subcore_axis_name="s")


@functools.partial(
    pl.kernel,
    out_type=jax.ShapeDtypeStruct((E,), jnp.float32),
    mesh=_mesh,
    compiler_params=pltpu.CompilerParams(needs_layout_passes=False,
                                         use_tc_tiling_on_sc=False),
    scratch_types=[
        pltpu.HBM((2, V, TW), jnp.int32),
        pltpu.VMEM((2, PCH, D), jnp.float32),
        pltpu.VMEM((2, PCH, TW), jnp.int32),
        pltpu.VMEM((EPW,), jnp.int32),
        pltpu.VMEM((EPW,), jnp.int32),
        pltpu.VMEM((EPW,), jnp.float32),
        pltpu.VMEM((2, C, TW), jnp.int32),
        pltpu.VMEM((2, C, TW), jnp.int32),
        pltpu.SemaphoreType.DMA,
        pltpu.SemaphoreType.DMA,
        pltpu.SemaphoreType.DMA,
        pltpu.SemaphoreType.DMA,
    ],
)
def _edge_sqdist(src_hbm, dst_hbm, table_hbm, out_hbm,
                 ptab, frows, prows, sidx, didx, outv, srows, drows,
                 sem0, sem1, semx, semo):
    cid = lax.axis_index("c")
    sid = lax.axis_index("s")
    w = sid * 2 + cid
    base = w * EPW
    sems = (sem0, sem1)

    pltpu.async_copy(src_hbm.at[pl.ds(base, EPW)], sidx, semx)
    pltpu.async_copy(dst_hbm.at[pl.ds(base, EPW)], didx, semx)

    rbase = sid * RPS

    def pin(ch, b):
        pltpu.async_copy(table_hbm.at[pl.ds(rbase + ch * PCH, PCH)],
                         frows.at[b], sems[b])

    def pin_wait(b):
        pltpu.make_async_copy(table_hbm.at[pl.ds(0, PCH)], frows.at[b],
                              sems[b]).wait()

    def pout_wait(b):
        pltpu.make_async_copy(ptab.at[0, pl.ds(0, PCH)], prows.at[b],
                              semo).wait()

    pin(0, 0)
    pin(1, 1)
    for ch in range(NPCH):
        b = ch % 2
        pin_wait(b)
        if ch >= 2:
            pout_wait(b)

        @plsc.parallel_loop(0, PCH, unroll=2)
        def row_body(r):
            for blk in range(D // 32):
                a = frows[b, r, pl.ds(blk * 32, 16)]
                bb = frows[b, r, pl.ds(blk * 32 + 16, 16)]
                pk = plsc.pack(a, bb, format=plsc.PackFormat.INTERLEAVED)
                prows[b, r, pl.ds(blk * 16, 16)] = plsc.bitcast(pk, jnp.int32)

        pltpu.async_copy(prows.at[b],
                         ptab.at[cid, pl.ds(rbase + ch * PCH, PCH)], semo)
        if ch + 2 < NPCH:
            pin(ch + 2, b)
    pout_wait(0)
    pout_wait(1 if NPCH > 1 else 0)
    plsc.subcore_barrier()

    pltpu.make_async_copy(src_hbm.at[pl.ds(0, EPW)], sidx, semx).wait()
    pltpu.make_async_copy(src_hbm.at[pl.ds(0, EPW)], didx, semx).wait()

    lane = lax.iota(jnp.int32, 16)

    def issue(k, b):
        off = pl.multiple_of(k * C, 8)
        pltpu.async_copy(ptab.at[cid].at[sidx.at[pl.ds(off, C)]],
                         srows.at[b], sems[b])
        pltpu.async_copy(ptab.at[cid].at[didx.at[pl.ds(off, C)]],
                         drows.at[b], sems[b])

    def drain(b):
        dummy = ptab.at[0, pl.ds(0, C)]
        pltpu.make_async_copy(dummy, srows.at[b], sems[b]).wait()
        pltpu.make_async_copy(dummy, drows.at[b], sems[b]).wait()

    def compute(k, b):
        off = pl.multiple_of(k * C, 8)
        sb = srows.at[b]
        db = drows.at[b]

        @plsc.parallel_loop(0, C // 16, unroll=1)
        def group_body(g):
            rows16 = g * 16 + lane

            def pair_body(t, acc):
                c0 = jnp.full((16,), 0, jnp.int32) + 2 * t
                c1 = c0 + 1
                sqs = []
                for cols in (c0, c1):
                    ws = plsc.load_gather(sb, [rows16, cols])
                    wd = plsc.load_gather(db, [rows16, cols])
                    dbf = plsc.bitcast(ws, jnp.bfloat16) - plsc.bitcast(
                        wd, jnp.bfloat16)
                    sqs.append(dbf * dbf)
                p0, p1 = plsc.unpack(sqs[0] + sqs[1],
                                     format=plsc.PackFormat.INTERLEAVED)
                return acc + p0 + p1

            tot = lax.fori_loop(0, TW // 2, pair_body,
                                jnp.zeros((16,), jnp.float32), unroll=4)
            outv[pl.ds(off + g * 16, 16)] = tot

    issue(0, 0)
    issue(1, 1)

    def pair_body(p, carry):
        for b in range(2):
            k = p * 2 + b
            drain(b)
            compute(k, b)

            @pl.when(k + 2 < NCHUNK)
            def _():
                issue(k + 2, b)
        return carry

    lax.fori_loop(0, NCHUNK // 2, pair_body, 0)
    drain(0)
    compute(NCHUNK - 1, 0)

    pltpu.sync_copy(outv, out_hbm.at[pl.ds(base, EPW)])


def kernel(src, dst, node_embed):
    src = src.astype(jnp.int32)
    dst = dst.astype(jnp.int32)
    return _edge_sqdist(src, dst, node_embed)


# speedup vs baseline: 1.0396x; 1.0396x over previous
import functools

import jax
import jax.numpy as jnp
from jax import lax
from jax.experimental import pallas as pl
from jax.experimental.pallas import tpu as pltpu
from jax.experimental.pallas import tpu_sc as plsc

E = 320000
D = 128
V = 10000
TW = D // 2
NW = 32
EPW = E // NW
C = 80
NCHUNK = EPW // C
RPS = V // 16
PCH = 125
NPCH = RPS // PCH

_mesh = plsc.VectorSubcoreMesh(core_axis_name="c", subcore_axis_name="s")


@functools.partial(
    pl.kernel,
    out_type=jax.ShapeDtypeStruct((E,), jnp.float32),
    mesh=_mesh,
    compiler_params=pltpu.CompilerParams(needs_layout_passes=False,
                                         use_tc_tiling_on_sc=False),
    scratch_types=[
        pltpu.HBM((2, V, TW), jnp.int32),
        pltpu.VMEM((2, PCH, D), jnp.float32),
        pltpu.VMEM((2, PCH, TW), jnp.int32),
        pltpu.VMEM((EPW,), jnp.int32),
        pltpu.VMEM((EPW,), jnp.int32),
        pltpu.VMEM((EPW,), jnp.float32),
        pltpu.VMEM((2, C, TW), jnp.int32),
        pltpu.VMEM((2, C, TW), jnp.int32),
        pltpu.SemaphoreType.DMA,
        pltpu.SemaphoreType.DMA,
        pltpu.SemaphoreType.DMA,
        pltpu.SemaphoreType.DMA,
    ],
)
def _edge_sqdist(src_hbm, dst_hbm, table_hbm, out_hbm,
                 ptab, frows, prows, sidx, didx, outv, srows, drows,
                 sem0, sem1, semx, semo):
    cid = lax.axis_index("c")
    sid = lax.axis_index("s")
    w = sid * 2 + cid
    base = w * EPW
    sems = (sem0, sem1)

    pltpu.async_copy(src_hbm.at[pl.ds(base, EPW)], sidx, semx)
    pltpu.async_copy(dst_hbm.at[pl.ds(base, EPW)], didx, semx)

    rbase = sid * RPS

    def pin(ch, b):
        pltpu.async_copy(table_hbm.at[pl.ds(rbase + ch * PCH, PCH)],
                         frows.at[b], sems[b])

    def pin_wait(b):
        pltpu.make_async_copy(table_hbm.at[pl.ds(0, PCH)], frows.at[b],
                              sems[b]).wait()

    def pout_wait(b):
        pltpu.make_async_copy(ptab.at[0, pl.ds(0, PCH)], prows.at[b],
                              semo).wait()

    pin(0, 0)
    pin(1, 1)
    for ch in range(NPCH):
        b = ch % 2
        pin_wait(b)
        if ch >= 2:
            pout_wait(b)

        @plsc.parallel_loop(0, PCH, unroll=2)
        def row_body(r):
            for blk in range(D // 32):
                a = frows[b, r, pl.ds(blk * 32, 16)]
                bb = frows[b, r, pl.ds(blk * 32 + 16, 16)]
                pk = plsc.pack(a, bb, format=plsc.PackFormat.INTERLEAVED)
                prows[b, r, pl.ds(blk * 16, 16)] = plsc.bitcast(pk, jnp.int32)

        pltpu.async_copy(prows.at[b],
                         ptab.at[cid, pl.ds(rbase + ch * PCH, PCH)], semo)
        if ch + 2 < NPCH:
            pin(ch + 2, b)
    pout_wait(0)
    pout_wait(1 if NPCH > 1 else 0)
    plsc.subcore_barrier()

    pltpu.make_async_copy(src_hbm.at[pl.ds(0, EPW)], sidx, semx).wait()
    pltpu.make_async_copy(src_hbm.at[pl.ds(0, EPW)], didx, semx).wait()

    lane = lax.iota(jnp.int32, 16)

    def issue(k, b):
        off = pl.multiple_of(k * C, 8)
        pltpu.async_copy(ptab.at[cid].at[sidx.at[pl.ds(off, C)]],
                         srows.at[b], sems[b])
        pltpu.async_copy(ptab.at[cid].at[didx.at[pl.ds(off, C)]],
                         drows.at[b], sems[b])

    def drain(b):
        dummy = ptab.at[0, pl.ds(0, C)]
        pltpu.make_async_copy(dummy, srows.at[b], sems[b]).wait()
        pltpu.make_async_copy(dummy, drows.at[b], sems[b]).wait()

    def compute(k, b):
        off = pl.multiple_of(k * C, 8)
        sb = srows.at[b]
        db = drows.at[b]

        @plsc.parallel_loop(0, C // 16, unroll=1)
        def group_body(g):
            rows16 = g * 16 + lane
            tot = jnp.zeros((16,), jnp.float32)
            for t in range(TW // 2):
                sqs = []
                for j in (2 * t, 2 * t + 1):
                    cols = jnp.full((16,), j, jnp.int32)
                    ws = plsc.load_gather(sb, [rows16, cols])
                    wd = plsc.load_gather(db, [rows16, cols])
                    dbf = plsc.bitcast(ws, jnp.bfloat16) - plsc.bitcast(
                        wd, jnp.bfloat16)
                    sqs.append(dbf * dbf)
                p0, p1 = plsc.unpack(sqs[0] + sqs[1],
                                     format=plsc.PackFormat.INTERLEAVED)
                tot = tot + p0 + p1
            outv[pl.ds(off + g * 16, 16)] = tot

    issue(0, 0)
    issue(1, 1)

    def pair_body(p, carry):
        for b in range(2):
            k = p * 2 + b
            drain(b)
            compute(k, b)

            @pl.when(k + 2 < NCHUNK)
            def _():
                issue(k + 2, b)
        return carry

    lax.fori_loop(0, NCHUNK // 2, pair_body, 0)
    drain(0)
    compute(NCHUNK - 1, 0)

    pltpu.sync_copy(outv, out_hbm.at[pl.ds(base, EPW)])


def kernel(src, dst, node_embed):
    src = src.astype(jnp.int32)
    dst = dst.astype(jnp.int32)
    return _edge_sqdist(src, dst, node_embed)


# speedup vs baseline: 4.9414x; 4.7534x over previous
import functools

import jax
import jax.numpy as jnp
from jax import lax
from jax.experimental import pallas as pl
from jax.experimental.pallas import tpu as pltpu
from jax.experimental.pallas import tpu_sc as plsc

E = 320000
D = 128
V = 10000
TW = D // 2
NW = 32
EPW = E // NW
C = 80
NCHUNK = EPW // C
RPS = V // 16
PCH = 125
NPCH = RPS // PCH

_mesh = plsc.VectorSubcoreMesh(core_axis_name="c", subcore_axis_name="s")


@functools.partial(
    pl.kernel,
    out_type=jax.ShapeDtypeStruct((E,), jnp.float32),
    mesh=_mesh,
    compiler_params=pltpu.CompilerParams(needs_layout_passes=False,
                                         use_tc_tiling_on_sc=False),
    scratch_types=[
        pltpu.HBM((2, V, TW), jnp.int32),
        pltpu.VMEM((2, PCH, D), jnp.float32),
        pltpu.VMEM((2, PCH, TW), jnp.int32),
        pltpu.VMEM((EPW,), jnp.int32),
        pltpu.VMEM((EPW,), jnp.int32),
        pltpu.VMEM((EPW,), jnp.float32),
        pltpu.VMEM((2, C, TW), jnp.int32),
        pltpu.VMEM((2, C, TW), jnp.int32),
        pltpu.VMEM((C * 16,), jnp.float32),
        pltpu.SemaphoreType.DMA,
        pltpu.SemaphoreType.DMA,
        pltpu.SemaphoreType.DMA,
        pltpu.SemaphoreType.DMA,
    ],
)
def _edge_sqdist(src_hbm, dst_hbm, table_hbm, out_hbm,
                 ptab, frows, prows, sidx, didx, outv, srows, drows, pv,
                 sem0, sem1, semx, semo):
    cid = lax.axis_index("c")
    sid = lax.axis_index("s")
    w = sid * 2 + cid
    base = w * EPW
    sems = (sem0, sem1)

    pltpu.async_copy(src_hbm.at[pl.ds(base, EPW)], sidx, semx)
    pltpu.async_copy(dst_hbm.at[pl.ds(base, EPW)], didx, semx)

    rbase = sid * RPS

    def pin(ch, b):
        pltpu.async_copy(table_hbm.at[pl.ds(rbase + ch * PCH, PCH)],
                         frows.at[b], sems[b])

    def pin_wait(b):
        pltpu.make_async_copy(table_hbm.at[pl.ds(0, PCH)], frows.at[b],
                              sems[b]).wait()

    def pout_wait(b):
        pltpu.make_async_copy(ptab.at[0, pl.ds(0, PCH)], prows.at[b],
                              semo).wait()

    pin(0, 0)
    pin(1, 1)
    for ch in range(NPCH):
        b = ch % 2
        pin_wait(b)
        if ch >= 2:
            pout_wait(b)

        @plsc.parallel_loop(0, PCH, unroll=2)
        def row_body(r):
            for blk in range(D // 32):
                a = frows[b, r, pl.ds(blk * 32, 16)]
                bb = frows[b, r, pl.ds(blk * 32 + 16, 16)]
                pk = plsc.pack(a, bb, format=plsc.PackFormat.INTERLEAVED)
                prows[b, r, pl.ds(blk * 16, 16)] = plsc.bitcast(pk, jnp.int32)

        pltpu.async_copy(prows.at[b],
                         ptab.at[cid, pl.ds(rbase + ch * PCH, PCH)], semo)
        if ch + 2 < NPCH:
            pin(ch + 2, b)
    pout_wait(0)
    pout_wait(1 if NPCH > 1 else 0)
    plsc.subcore_barrier()

    pltpu.make_async_copy(src_hbm.at[pl.ds(0, EPW)], sidx, semx).wait()
    pltpu.make_async_copy(src_hbm.at[pl.ds(0, EPW)], didx, semx).wait()

    lane = lax.iota(jnp.int32, 16)

    def issue(k, b):
        off = pl.multiple_of(k * C, 8)
        pltpu.async_copy(ptab.at[cid].at[sidx.at[pl.ds(off, C)]],
                         srows.at[b], sems[b])
        pltpu.async_copy(ptab.at[cid].at[didx.at[pl.ds(off, C)]],
                         drows.at[b], sems[b])

    def drain(b):
        dummy = ptab.at[0, pl.ds(0, C)]
        pltpu.make_async_copy(dummy, srows.at[b], sems[b]).wait()
        pltpu.make_async_copy(dummy, drows.at[b], sems[b]).wait()

    def compute(k, b):
        off = pl.multiple_of(k * C, 8)
        sb = srows.at[b]
        db = drows.at[b]

        @plsc.parallel_loop(0, C, unroll=4)
        def edge_body(i):
            acc = jnp.zeros((16,), jnp.float32)
            for half in range(TW // 32):
                sqs = []
                for kk in (2 * half, 2 * half + 1):
                    ws = sb[i, pl.ds(kk * 16, 16)]
                    wd = db[i, pl.ds(kk * 16, 16)]
                    dbf = plsc.bitcast(ws, jnp.bfloat16) - plsc.bitcast(
                        wd, jnp.bfloat16)
                    sqs.append(dbf * dbf)
                p0, p1 = plsc.unpack(sqs[0] + sqs[1],
                                     format=plsc.PackFormat.INTERLEAVED)
                acc = acc + p0 + p1
            pv[pl.ds(i * 16, 16)] = acc

        @plsc.parallel_loop(0, C // 16, unroll=1)
        def group_body(g):
            rowbase = g * 256 + lane * 16
            tot = jnp.zeros((16,), jnp.float32)
            for kk in range(16):
                tot = tot + plsc.load_gather(pv, [rowbase + kk])
            outv[pl.ds(off + g * 16, 16)] = tot

    issue(0, 0)
    issue(1, 1)

    def pair_body(p, carry):
        for b in range(2):
            k = p * 2 + b
            drain(b)
            compute(k, b)

            @pl.when(k + 2 < NCHUNK)
            def _():
                issue(k + 2, b)
        return carry

    lax.fori_loop(0, NCHUNK // 2, pair_body, 0)
    drain(0)
    compute(NCHUNK - 1, 0)

    pltpu.sync_copy(outv, out_hbm.at[pl.ds(base, EPW)])


def kernel(src, dst, node_embed):
    src = src.astype(jnp.int32)
    dst = dst.astype(jnp.int32)
    return _edge_sqdist(src, dst, node_embed)


# speedup vs baseline: 5.7922x; 1.1722x over previous
import functools

import jax
import jax.numpy as jnp
from jax import lax
from jax.experimental import pallas as pl
from jax.experimental.pallas import tpu as pltpu
from jax.experimental.pallas import tpu_sc as plsc

E = 320000
D = 128
V = 10000
TW = D // 2
NW = 32
EPW = E // NW
C = 160
G = 80
NCHUNK = EPW // C
TAIL = EPW - NCHUNK * C
RPS = V // 16
PCH = 125
NPCH = RPS // PCH

_mesh = plsc.VectorSubcoreMesh(core_axis_name="c", subcore_axis_name="s")


@functools.partial(
    pl.kernel,
    out_type=jax.ShapeDtypeStruct((E,), jnp.float32),
    mesh=_mesh,
    compiler_params=pltpu.CompilerParams(needs_layout_passes=False,
                                         use_tc_tiling_on_sc=False),
    scratch_types=[
        pltpu.HBM((2, V, TW), jnp.int32),
        pltpu.VMEM((2, PCH, D), jnp.float32),
        pltpu.VMEM((2, PCH, TW), jnp.int32),
        pltpu.VMEM((EPW,), jnp.int32),
        pltpu.VMEM((EPW,), jnp.int32),
        pltpu.VMEM((EPW,), jnp.float32),
        pltpu.VMEM((2, C, TW), jnp.int32),
        pltpu.VMEM((2, C, TW), jnp.int32),
        pltpu.VMEM((C * 16,), jnp.float32),
        pltpu.SemaphoreType.DMA,
        pltpu.SemaphoreType.DMA,
        pltpu.SemaphoreType.DMA,
        pltpu.SemaphoreType.DMA,
    ],
)
def _edge_sqdist(src_hbm, dst_hbm, table_hbm, out_hbm,
                 ptab, frows, prows, sidx, didx, outv, srows, drows, pv,
                 sem0, sem1, semx, semo):
    cid = lax.axis_index("c")
    sid = lax.axis_index("s")
    w = sid * 2 + cid
    base = w * EPW
    sems = (sem0, sem1)

    pltpu.async_copy(src_hbm.at[pl.ds(base, EPW)], sidx, semx)
    pltpu.async_copy(dst_hbm.at[pl.ds(base, EPW)], didx, semx)

    rbase = sid * RPS

    def pin(ch, b):
        pltpu.async_copy(table_hbm.at[pl.ds(rbase + ch * PCH, PCH)],
                         frows.at[b], sems[b])

    def pin_wait(b):
        pltpu.make_async_copy(table_hbm.at[pl.ds(0, PCH)], frows.at[b],
                              sems[b]).wait()

    def pout_wait(b):
        pltpu.make_async_copy(ptab.at[0, pl.ds(0, PCH)], prows.at[b],
                              semo).wait()

    pin(0, 0)
    pin(1, 1)
    for ch in range(NPCH):
        b = ch % 2
        pin_wait(b)
        if ch >= 2:
            pout_wait(b)

        @plsc.parallel_loop(0, PCH, unroll=2)
        def row_body(r):
            for blk in range(D // 32):
                a = frows[b, r, pl.ds(blk * 32, 16)]
                bb = frows[b, r, pl.ds(blk * 32 + 16, 16)]
                pk = plsc.pack(a, bb, format=plsc.PackFormat.INTERLEAVED)
                prows[b, r, pl.ds(blk * 16, 16)] = plsc.bitcast(pk, jnp.int32)

        pltpu.async_copy(prows.at[b],
                         ptab.at[cid, pl.ds(rbase + ch * PCH, PCH)], semo)
        if ch + 2 < NPCH:
            pin(ch + 2, b)
    pout_wait(0)
    pout_wait(1 if NPCH > 1 else 0)
    plsc.subcore_barrier()

    pltpu.make_async_copy(src_hbm.at[pl.ds(0, EPW)], sidx, semx).wait()
    pltpu.make_async_copy(src_hbm.at[pl.ds(0, EPW)], didx, semx).wait()

    lane = lax.iota(jnp.int32, 16)

    def issue(off, n, b):
        for sub in range(n // G):
            so = off + sub * G
            pltpu.async_copy(ptab.at[cid].at[sidx.at[pl.ds(so, G)]],
                             srows.at[b, pl.ds(sub * G, G)], sems[b])
            pltpu.async_copy(ptab.at[cid].at[didx.at[pl.ds(so, G)]],
                             drows.at[b, pl.ds(sub * G, G)], sems[b])

    def drain(n, b):
        dummy = ptab.at[0, pl.ds(0, n)]
        pltpu.make_async_copy(dummy, srows.at[b, pl.ds(0, n)],
                              sems[b]).wait()
        pltpu.make_async_copy(dummy, drows.at[b, pl.ds(0, n)],
                              sems[b]).wait()

    def compute(off, n, b):
        sb = srows.at[b]
        db = drows.at[b]

        @plsc.parallel_loop(0, n, unroll=4)
        def edge_body(i):
            acc = jnp.zeros((16,), jnp.float32)
            for half in range(TW // 32):
                sqs = []
                for kk in (2 * half, 2 * half + 1):
                    ws = sb[i, pl.ds(kk * 16, 16)]
                    wd = db[i, pl.ds(kk * 16, 16)]
                    dbf = plsc.bitcast(ws, jnp.bfloat16) - plsc.bitcast(
                        wd, jnp.bfloat16)
                    sqs.append(dbf * dbf)
                p0, p1 = plsc.unpack(sqs[0] + sqs[1],
                                     format=plsc.PackFormat.INTERLEAVED)
                acc = acc + p0 + p1
            pv[pl.ds(i * 16, 16)] = acc

        @plsc.parallel_loop(0, n // 16, unroll=1)
        def group_body(g):
            rowbase = g * 256 + lane * 16
            tot = jnp.zeros((16,), jnp.float32)
            for kk in range(16):
                tot = tot + plsc.load_gather(pv, [rowbase + kk])
            outv[pl.ds(off + g * 16, 16)] = tot

    issue(0, C, 0)
    issue(C, C, 1)

    def pair_body(p, carry):
        for b in range(2):
            k = p * 2 + b
            drain(C, b)
            compute(pl.multiple_of(k * C, 8), C, b)

            @pl.when(k + 2 < NCHUNK)
            def _():
                issue(pl.multiple_of((k + 2) * C, 8), C, b)

            @pl.when(k + 2 == NCHUNK)
            def _():
                issue(NCHUNK * C, TAIL, b)
        return carry

    lax.fori_loop(0, NCHUNK // 2, pair_body, 0)
    drain(TAIL, 0)
    compute(NCHUNK * C, TAIL, 0)

    pltpu.sync_copy(outv, out_hbm.at[pl.ds(base, EPW)])


def kernel(src, dst, node_embed):
    src = src.astype(jnp.int32)
    dst = dst.astype(jnp.int32)
    return _edge_sqdist(src, dst, node_embed)
